# baseline (speedup 1.0000x reference)
import jax
import jax.numpy as jnp
from jax.experimental import pallas as pl
from jax.experimental.pallas import tpu as pltpu

N = 10000
CSPLIT = 4864
BI = 400
BI_MID = 1024
NHID = 64
NCLASS = 40
F8 = jnp.float8_e4m3fn


def _gc_first_kernel(h_ref, W_ref, b_ref, adj_ref, out_ref, adjb_ref, adj8_ref, sup_ref):
    @pl.when(pl.program_id(0) == 0)
    def _():
        sup_ref[...] = jnp.dot(
            h_ref[...], W_ref[...], preferred_element_type=jnp.float32
        )

    a = adj_ref[...]
    adjb_ref[...] = a[:, :CSPLIT].astype(jnp.bfloat16)
    adj8_ref[...] = a[:, CSPLIT:].astype(F8)
    out_ref[...] = (
        jnp.dot(a, sup_ref[...], preferred_element_type=jnp.float32) + b_ref[...]
    )


def _gc_first_layer(h, W, b, adj):
    din, dout = W.shape
    return pl.pallas_call(
        _gc_first_kernel,
        grid=(N // BI,),
        in_specs=[
            pl.BlockSpec((N, din), lambda i: (0, 0)),
            pl.BlockSpec((din, dout), lambda i: (0, 0)),
            pl.BlockSpec((1, dout), lambda i: (0, 0)),
            pl.BlockSpec((BI, N), lambda i: (i, 0)),
        ],
        out_specs=[
            pl.BlockSpec((BI, dout), lambda i: (i, 0)),
            pl.BlockSpec((BI, CSPLIT), lambda i: (i, 0)),
            pl.BlockSpec((BI, N - CSPLIT), lambda i: (i, 0)),
        ],
        out_shape=[
            jax.ShapeDtypeStruct((N, dout), jnp.float32),
            jax.ShapeDtypeStruct((N, CSPLIT), jnp.bfloat16),
            jax.ShapeDtypeStruct((N, N - CSPLIT), F8),
        ],
        scratch_shapes=[pltpu.VMEM((N, dout), jnp.float32)],
    )(h, W, b, adj)


def _mid_kernel(e1_ref, W_ref, b_ref, adjb_ref, adj8_ref, emb_ref, ls_ref, sup_ref, h_ref):
    l = pl.program_id(0)
    j = pl.program_id(1)

    @pl.when(j == 0)
    def _():
        h = jnp.where(l == 0, e1_ref[...], h_ref[:N, :])
        h = jnp.maximum(h, 0.0)
        sup_ref[...] = jnp.dot(
            h, W_ref[0], preferred_element_type=jnp.float32
        ).astype(jnp.bfloat16)

    eb = jnp.dot(
        adjb_ref[...], sup_ref[:CSPLIT, :], preferred_element_type=jnp.float32
    )
    e8 = jnp.dot(
        adj8_ref[...].astype(jnp.bfloat16),
        sup_ref[CSPLIT:, :],
        preferred_element_type=jnp.float32,
    )
    e = eb + e8 + b_ref[0]
    emb_ref[0] = e
    h_ref[pl.ds(j * BI_MID, BI_MID), :] = e

    @pl.when(l == 3)
    def _():
        cols = jax.lax.broadcasted_iota(jnp.int32, e.shape, 1)
        em = jnp.where(cols < NCLASS, e, -jnp.inf)
        m = jnp.max(em, axis=1, keepdims=True)
        lse = jnp.log(jnp.sum(jnp.exp(em - m), axis=1, keepdims=True)) + m
        ls_ref[...] = e - lse


def _mid_layers(e1, Wpack, bpack, adjb, adj8):
    nj = pl.cdiv(N, BI_MID)
    return pl.pallas_call(
        _mid_kernel,
        grid=(4, nj),
        in_specs=[
            pl.BlockSpec((N, NHID), lambda l, j: (0, 0)),
            pl.BlockSpec((1, NHID, NHID), lambda l, j: (l, 0, 0)),
            pl.BlockSpec((1, 1, NHID), lambda l, j: (l, 0, 0)),
            pl.BlockSpec((BI_MID, CSPLIT), lambda l, j: (j, 0)),
            pl.BlockSpec((BI_MID, N - CSPLIT), lambda l, j: (j, 0)),
        ],
        out_specs=[
            pl.BlockSpec((1, BI_MID, NHID), lambda l, j: (l, j, 0)),
            pl.BlockSpec((BI_MID, NHID), lambda l, j: (j, 0)),
        ],
        out_shape=[
            jax.ShapeDtypeStruct((4, N, NHID), jnp.float32),
            jax.ShapeDtypeStruct((N, NHID), jnp.float32),
        ],
        scratch_shapes=[
            pltpu.VMEM((N, NHID), jnp.bfloat16),
            pltpu.VMEM((nj * BI_MID, NHID), jnp.float32),
        ],
    )(e1, Wpack, bpack, adjb, adj8)


def kernel(x, adj, W1, b1, W2, b2, W3, b3, W4, b4, W5, b5):
    b1r = b1.reshape(1, -1)
    Wpack = jnp.stack(
        [
            W2,
            W3,
            W4,
            jnp.pad(W5, ((0, 0), (0, NHID - NCLASS))),
        ]
    )
    bpack = jnp.stack(
        [
            b2.reshape(1, -1),
            b3.reshape(1, -1),
            b4.reshape(1, -1),
            jnp.pad(b5, (0, NHID - NCLASS)).reshape(1, -1),
        ]
    )
    e1, adjb, adj8 = _gc_first_layer(x, W1, b1r, adj)
    embs, ls = _mid_layers(e1, Wpack, bpack, adjb, adj8)
    e2, e3, e4 = embs[0], embs[1], embs[2]
    e5 = embs[3, :, :NCLASS]
    out = ls[:, :NCLASS]
    return (out, e1, e2, e3, e4, e5)


# speedup vs baseline: 1.0449x; 1.0449x over previous
import jax
import jax.numpy as jnp
from jax.experimental import pallas as pl
from jax.experimental.pallas import tpu as pltpu

N = 10000
BI = 400
BI_MID = 512
NHID = 64
NCLASS = 40
F8 = jnp.float8_e4m3fn


def _gc_first_kernel(h_ref, W_ref, b_ref, adj_ref, out_ref, adj8_ref, sup_ref):
    @pl.when(pl.program_id(0) == 0)
    def _():
        sup_ref[...] = jnp.dot(
            h_ref[...], W_ref[...], preferred_element_type=jnp.float32
        )

    a = adj_ref[...]
    adj8_ref[...] = a.astype(F8)
    out_ref[...] = (
        jnp.dot(a, sup_ref[...], preferred_element_type=jnp.float32) + b_ref[...]
    )


def _gc_first_layer(h, W, b, adj):
    din, dout = W.shape
    return pl.pallas_call(
        _gc_first_kernel,
        grid=(N // BI,),
        in_specs=[
            pl.BlockSpec((N, din), lambda i: (0, 0)),
            pl.BlockSpec((din, dout), lambda i: (0, 0)),
            pl.BlockSpec((1, dout), lambda i: (0, 0)),
            pl.BlockSpec((BI, N), lambda i: (i, 0)),
        ],
        out_specs=[
            pl.BlockSpec((BI, dout), lambda i: (i, 0)),
            pl.BlockSpec((BI, N), lambda i: (i, 0)),
        ],
        out_shape=[
            jax.ShapeDtypeStruct((N, dout), jnp.float32),
            jax.ShapeDtypeStruct((N, N), F8),
        ],
        scratch_shapes=[pltpu.VMEM((N, dout), jnp.float32)],
    )(h, W, b, adj)


def _mid_kernel(e1_ref, W_ref, b_ref, adj8_ref, emb_ref, ls_ref, sup_ref, h_ref):
    l = pl.program_id(0)
    j = pl.program_id(1)

    @pl.when(j == 0)
    def _():
        h = jnp.where(l == 0, e1_ref[...], h_ref[:N, :])
        h = jnp.maximum(h, 0.0)
        sup_ref[...] = jnp.dot(
            h, W_ref[0], preferred_element_type=jnp.float32
        ).astype(jnp.bfloat16)

    a = adj8_ref[...].astype(jnp.bfloat16)
    e = jnp.dot(a, sup_ref[...], preferred_element_type=jnp.float32) + b_ref[0]
    emb_ref[0] = e
    h_ref[pl.ds(j * BI_MID, BI_MID), :] = e

    @pl.when(l == 3)
    def _():
        cols = jax.lax.broadcasted_iota(jnp.int32, e.shape, 1)
        em = jnp.where(cols < NCLASS, e, -jnp.inf)
        m = jnp.max(em, axis=1, keepdims=True)
        lse = jnp.log(jnp.sum(jnp.exp(em - m), axis=1, keepdims=True)) + m
        ls_ref[...] = e - lse


def _mid_layers(e1, Wpack, bpack, adj8):
    nj = pl.cdiv(N, BI_MID)
    return pl.pallas_call(
        _mid_kernel,
        grid=(4, nj),
        in_specs=[
            pl.BlockSpec((N, NHID), lambda l, j: (0, 0)),
            pl.BlockSpec((1, NHID, NHID), lambda l, j: (l, 0, 0)),
            pl.BlockSpec((1, 1, NHID), lambda l, j: (l, 0, 0)),
            pl.BlockSpec((BI_MID, N), lambda l, j: (j, 0)),
        ],
        out_specs=[
            pl.BlockSpec((1, BI_MID, NHID), lambda l, j: (l, j, 0)),
            pl.BlockSpec((BI_MID, NHID), lambda l, j: (j, 0)),
        ],
        out_shape=[
            jax.ShapeDtypeStruct((4, N, NHID), jnp.float32),
            jax.ShapeDtypeStruct((N, NHID), jnp.float32),
        ],
        scratch_shapes=[
            pltpu.VMEM((N, NHID), jnp.bfloat16),
            pltpu.VMEM((nj * BI_MID, NHID), jnp.float32),
        ],
    )(e1, Wpack, bpack, adj8)


def kernel(x, adj, W1, b1, W2, b2, W3, b3, W4, b4, W5, b5):
    b1r = b1.reshape(1, -1)
    Wpack = jnp.stack(
        [
            W2,
            W3,
            W4,
            jnp.pad(W5, ((0, 0), (0, NHID - NCLASS))),
        ]
    )
    bpack = jnp.stack(
        [
            b2.reshape(1, -1),
            b3.reshape(1, -1),
            b4.reshape(1, -1),
            jnp.pad(b5, (0, NHID - NCLASS)).reshape(1, -1),
        ]
    )
    e1, adj8 = _gc_first_layer(x, W1, b1r, adj)
    embs, ls = _mid_layers(e1, Wpack, bpack, adj8)
    e2, e3, e4 = embs[0], embs[1], embs[2]
    e5 = embs[3, :, :NCLASS]
    out = ls[:, :NCLASS]
    return (out, e1, e2, e3, e4, e5)


# speedup vs baseline: 1.0791x; 1.0327x over previous
import jax
import jax.numpy as jnp
from jax.experimental import pallas as pl
from jax.experimental.pallas import tpu as pltpu

N = 10000
BI = 400
BI_MID = 1024
NHID = 64
NCLASS = 40
F8 = jnp.float8_e4m3fn


def _gc_first_kernel(h_ref, W_ref, b_ref, adj_ref, out_ref, adj8_ref, sup_ref):
    @pl.when(pl.program_id(0) == 0)
    def _():
        sup_ref[...] = jnp.dot(
            h_ref[...], W_ref[...], preferred_element_type=jnp.float32
        )

    a = adj_ref[...]
    adj8_ref[...] = a.astype(F8)
    out_ref[...] = (
        jnp.dot(a, sup_ref[...], preferred_element_type=jnp.float32) + b_ref[...]
    )


def _gc_first_layer(h, W, b, adj):
    din, dout = W.shape
    return pl.pallas_call(
        _gc_first_kernel,
        grid=(N // BI,),
        in_specs=[
            pl.BlockSpec((N, din), lambda i: (0, 0)),
            pl.BlockSpec((din, dout), lambda i: (0, 0)),
            pl.BlockSpec((1, dout), lambda i: (0, 0)),
            pl.BlockSpec((BI, N), lambda i: (i, 0)),
        ],
        out_specs=[
            pl.BlockSpec((BI, dout), lambda i: (i, 0)),
            pl.BlockSpec((BI, N), lambda i: (i, 0)),
        ],
        out_shape=[
            jax.ShapeDtypeStruct((N, dout), jnp.float32),
            jax.ShapeDtypeStruct((N, N), F8),
        ],
        scratch_shapes=[pltpu.VMEM((N, dout), jnp.float32)],
    )(h, W, b, adj)


def _mid_kernel(e1_ref, W_ref, b_ref, adj8_ref, emb_ref, ls_ref, sup_ref, h_ref):
    l = pl.program_id(0)
    j = pl.program_id(1)

    @pl.when(j == 0)
    def _():
        h = jnp.where(l == 0, e1_ref[...], h_ref[:N, :])
        h = jnp.maximum(h, 0.0)
        sup_ref[...] = jnp.dot(
            h, W_ref[0], preferred_element_type=jnp.float32
        ).astype(jnp.bfloat16)

    a = adj8_ref[...].astype(jnp.bfloat16)
    e = jnp.dot(a, sup_ref[...], preferred_element_type=jnp.float32) + b_ref[0]
    emb_ref[0] = e
    h_ref[pl.ds(j * BI_MID, BI_MID), :] = e

    @pl.when(l == 3)
    def _():
        cols = jax.lax.broadcasted_iota(jnp.int32, e.shape, 1)
        em = jnp.where(cols < NCLASS, e, -jnp.inf)
        m = jnp.max(em, axis=1, keepdims=True)
        lse = jnp.log(jnp.sum(jnp.exp(em - m), axis=1, keepdims=True)) + m
        ls_ref[...] = e - lse


def _mid_layers(e1, Wpack, bpack, adj8):
    nj = pl.cdiv(N, BI_MID)
    return pl.pallas_call(
        _mid_kernel,
        grid=(4, nj),
        in_specs=[
            pl.BlockSpec((N, NHID), lambda l, j: (0, 0)),
            pl.BlockSpec((1, NHID, NHID), lambda l, j: (l, 0, 0)),
            pl.BlockSpec((1, 1, NHID), lambda l, j: (l, 0, 0)),
            pl.BlockSpec((BI_MID, N), lambda l, j: (j, 0)),
        ],
        out_specs=[
            pl.BlockSpec((1, BI_MID, NHID), lambda l, j: (l, j, 0)),
            pl.BlockSpec((BI_MID, NHID), lambda l, j: (j, 0)),
        ],
        out_shape=[
            jax.ShapeDtypeStruct((4, N, NHID), jnp.float32),
            jax.ShapeDtypeStruct((N, NHID), jnp.float32),
        ],
        scratch_shapes=[
            pltpu.VMEM((N, NHID), jnp.bfloat16),
            pltpu.VMEM((nj * BI_MID, NHID), jnp.float32),
        ],
    )(e1, Wpack, bpack, adj8)


def kernel(x, adj, W1, b1, W2, b2, W3, b3, W4, b4, W5, b5):
    b1r = b1.reshape(1, -1)
    Wpack = jnp.stack(
        [
            W2,
            W3,
            W4,
            jnp.pad(W5, ((0, 0), (0, NHID - NCLASS))),
        ]
    )
    bpack = jnp.stack(
        [
            b2.reshape(1, -1),
            b3.reshape(1, -1),
            b4.reshape(1, -1),
            jnp.pad(b5, (0, NHID - NCLASS)).reshape(1, -1),
        ]
    )
    e1, adj8 = _gc_first_layer(x, W1, b1r, adj)
    embs, ls = _mid_layers(e1, Wpack, bpack, adj8)
    e2, e3, e4 = embs[0], embs[1], embs[2]
    e5 = embs[3, :, :NCLASS]
    out = ls[:, :NCLASS]
    return (out, e1, e2, e3, e4, e5)
